# baseline (device time: 77232 ns/iter reference)
import jax
import jax.numpy as jnp
from jax import lax
from jax.experimental import pallas as pl
from jax.experimental.pallas import tpu as pltpu

N_DEV = 4
SEQ = 1024
CHUNK = 256
D = 1024
HEADS = 8
DH = 128
SCALE = 0.08838834764831843


def kernel(x, Wq, Wo, Wk, Wv):
    x2 = x.reshape(CHUNK, D)

    def body(x_ref, wq_ref, wo_ref, wk_ref, wv_ref, out_ref,
             xfull, ystore, rs_buf,
             ag_send_sems, ag_recv_sems, rs_send_sems, rs_recv_sems):
        my = lax.axis_index("i")
        left = (my - 1) % N_DEV
        right = (my + 1) % N_DEV

        barrier_sem = pltpu.get_barrier_semaphore()
        for nbr in (left, right):
            pl.semaphore_signal(barrier_sem, inc=1, device_id=(nbr,),
                                device_id_type=pl.DeviceIdType.MESH)
        pl.semaphore_wait(barrier_sem, 2)

        xfull[pl.ds(my * CHUNK, CHUNK), :] = x_ref[...].astype(jnp.bfloat16)
        for h in range(N_DEV - 1):
            src_slot = (my - h) % N_DEV
            rdma = pltpu.make_async_remote_copy(
                src_ref=xfull.at[pl.ds(src_slot * CHUNK, CHUNK)],
                dst_ref=xfull.at[pl.ds(src_slot * CHUNK, CHUNK)],
                send_sem=ag_send_sems.at[h],
                recv_sem=ag_recv_sems.at[h],
                device_id=(right,),
                device_id_type=pl.DeviceIdType.MESH,
            )
            rdma.start()
            rdma.wait()

        xf = xfull[...]
        wq = wq_ref[...].astype(jnp.bfloat16)
        wk = wk_ref[...].astype(jnp.bfloat16)
        wv = wv_ref[...].astype(jnp.bfloat16)
        wo = wo_ref[...].astype(jnp.bfloat16)
        Q = jnp.dot(xf, wq, preferred_element_type=jnp.float32)
        K = jnp.dot(xf, wk, preferred_element_type=jnp.float32)
        V = jnp.dot(xf, wv, preferred_element_type=jnp.float32)

        ypart = jnp.zeros((SEQ, D), jnp.float32)
        for hh in range(HEADS):
            sl = slice(hh * DH, (hh + 1) * DH)
            qh = Q[:, sl].astype(jnp.bfloat16)
            kh = K[:, sl].astype(jnp.bfloat16)
            vh = V[:, sl].astype(jnp.bfloat16)
            s = lax.dot_general(
                qh, kh, (((1,), (1,)), ((), ())),
                preferred_element_type=jnp.float32) * SCALE
            m = jnp.max(s, axis=-1, keepdims=True)
            p = jnp.exp(s - m)
            l = jnp.sum(p, axis=-1, keepdims=True)
            o = jnp.dot(p.astype(jnp.bfloat16), vh,
                        preferred_element_type=jnp.float32) / l
            ypart = ypart + jnp.dot(o.astype(jnp.bfloat16), wo[sl, :],
                                    preferred_element_type=jnp.float32)
        ystore[...] = ypart.astype(jnp.bfloat16)

        rdmas = []
        for d in range(1, N_DEV):
            tgt = (my + d) % N_DEV
            rdma = pltpu.make_async_remote_copy(
                src_ref=ystore.at[pl.ds(tgt * CHUNK, CHUNK)],
                dst_ref=rs_buf.at[pl.ds(my * CHUNK, CHUNK)],
                send_sem=rs_send_sems.at[d - 1],
                recv_sem=rs_recv_sems.at[my],
                device_id=(tgt,),
                device_id_type=pl.DeviceIdType.MESH,
            )
            rdma.start()
            rdmas.append(rdma)

        rs_buf[pl.ds(my * CHUNK, CHUNK), :] = ystore[pl.ds(my * CHUNK, CHUNK), :]

        for p in range(N_DEV):
            @pl.when(p != my)
            def _():
                recv = pltpu.make_async_remote_copy(
                    src_ref=rs_buf.at[pl.ds(p * CHUNK, CHUNK)],
                    dst_ref=rs_buf.at[pl.ds(p * CHUNK, CHUNK)],
                    send_sem=rs_send_sems.at[0],
                    recv_sem=rs_recv_sems.at[p],
                    device_id=(p,),
                    device_id_type=pl.DeviceIdType.MESH,
                )
                recv.wait_recv()

        total = jnp.zeros((CHUNK, D), jnp.float32)
        for p in range(N_DEV):
            total = total + rs_buf[p * CHUNK:(p + 1) * CHUNK, :].astype(
                jnp.float32)
        out_ref[...] = total

        for rdma in rdmas:
            rdma.wait_send()

    out = pl.pallas_call(
        body,
        out_shape=jax.ShapeDtypeStruct((CHUNK, D), jnp.float32),
        in_specs=[pl.BlockSpec(memory_space=pltpu.VMEM)] * 5,
        out_specs=pl.BlockSpec(memory_space=pltpu.VMEM),
        scratch_shapes=[
            pltpu.VMEM((SEQ, D), jnp.bfloat16),
            pltpu.VMEM((SEQ, D), jnp.bfloat16),
            pltpu.VMEM((SEQ, D), jnp.bfloat16),
            pltpu.SemaphoreType.DMA((N_DEV - 1,)),
            pltpu.SemaphoreType.DMA((N_DEV - 1,)),
            pltpu.SemaphoreType.DMA((N_DEV - 1,)),
            pltpu.SemaphoreType.DMA((N_DEV,)),
        ],
        compiler_params=pltpu.CompilerParams(
            collective_id=0,
            vmem_limit_bytes=100 * 1024 * 1024,
        ),
    )(x2, Wq, Wo, Wk, Wv)
    return out.reshape(1, CHUNK, D)


# device time: 63793 ns/iter; 1.2107x vs baseline; 1.2107x over previous
import jax
import jax.numpy as jnp
from jax import lax
from jax.experimental import pallas as pl
from jax.experimental.pallas import tpu as pltpu

N_DEV = 4
SEQ = 1024
CHUNK = 256
D = 1024
HEADS = 8
DH = 128
SCALE = 0.08838834764831843


def kernel(x, Wq, Wo, Wk, Wv):
    x2 = x.reshape(CHUNK, D)

    def body(x_ref, wq_ref, wo_ref, wk_ref, wv_ref, out_ref,
             xfull, qbuf, kbuf, vbuf, ystore, rs_buf,
             ag_send_sems, ag_recv_sems, rs_send_sems, rs_recv_sems):
        my = lax.axis_index("i")
        left = (my - 1) % N_DEV
        right = (my + 1) % N_DEV

        barrier_sem = pltpu.get_barrier_semaphore()
        for nbr in (left, right):
            pl.semaphore_signal(barrier_sem, inc=1, device_id=(nbr,),
                                device_id_type=pl.DeviceIdType.MESH)
        pl.semaphore_wait(barrier_sem, 2)

        wq = wq_ref[...].astype(jnp.bfloat16)
        wk = wk_ref[...].astype(jnp.bfloat16)
        wv = wv_ref[...].astype(jnp.bfloat16)
        wo = wo_ref[...].astype(jnp.bfloat16)

        def qkv_chunk(slot):
            xc = xfull[pl.ds(slot * CHUNK, CHUNK), :]
            qbuf[pl.ds(slot * CHUNK, CHUNK), :] = jnp.dot(
                xc, wq, preferred_element_type=jnp.float32
            ).astype(jnp.bfloat16)
            kbuf[pl.ds(slot * CHUNK, CHUNK), :] = jnp.dot(
                xc, wk, preferred_element_type=jnp.float32
            ).astype(jnp.bfloat16)
            vbuf[pl.ds(slot * CHUNK, CHUNK), :] = jnp.dot(
                xc, wv, preferred_element_type=jnp.float32
            ).astype(jnp.bfloat16)

        xfull[pl.ds(my * CHUNK, CHUNK), :] = x_ref[...].astype(jnp.bfloat16)
        ag_rdmas = []
        for h in range(N_DEV - 1):
            src_slot = (my - h) % N_DEV
            rdma = pltpu.make_async_remote_copy(
                src_ref=xfull.at[pl.ds(src_slot * CHUNK, CHUNK)],
                dst_ref=xfull.at[pl.ds(src_slot * CHUNK, CHUNK)],
                send_sem=ag_send_sems.at[h],
                recv_sem=ag_recv_sems.at[h],
                device_id=(right,),
                device_id_type=pl.DeviceIdType.MESH,
            )
            rdma.start()
            ag_rdmas.append(rdma)
            qkv_chunk(src_slot)
            rdma.wait_recv()
        qkv_chunk((my - (N_DEV - 1)) % N_DEV)

        def attn_chunk(c):
            qrows = qbuf[pl.ds(c * CHUNK, CHUNK), :]
            acc = jnp.zeros((CHUNK, D), jnp.float32)
            for hh in range(HEADS):
                sl = slice(hh * DH, (hh + 1) * DH)
                qh = qrows[:, sl]
                kh = kbuf[:, sl]
                vh = vbuf[:, sl]
                s = lax.dot_general(
                    qh, kh, (((1,), (1,)), ((), ())),
                    preferred_element_type=jnp.float32) * SCALE
                m = jnp.max(s, axis=-1, keepdims=True)
                p = jnp.exp(s - m)
                l = jnp.sum(p, axis=-1, keepdims=True)
                o = jnp.dot(p.astype(jnp.bfloat16), vh,
                            preferred_element_type=jnp.float32) / l
                acc = acc + jnp.dot(o.astype(jnp.bfloat16), wo[sl, :],
                                    preferred_element_type=jnp.float32)
            ystore[pl.ds(c * CHUNK, CHUNK), :] = acc.astype(jnp.bfloat16)

        rs_rdmas = []
        for d in range(1, N_DEV):
            tgt = (my + d) % N_DEV
            attn_chunk(tgt)
            rdma = pltpu.make_async_remote_copy(
                src_ref=ystore.at[pl.ds(tgt * CHUNK, CHUNK)],
                dst_ref=rs_buf.at[pl.ds(my * CHUNK, CHUNK)],
                send_sem=rs_send_sems.at[d - 1],
                recv_sem=rs_recv_sems.at[my],
                device_id=(tgt,),
                device_id_type=pl.DeviceIdType.MESH,
            )
            rdma.start()
            rs_rdmas.append(rdma)

        attn_chunk(my)
        rs_buf[pl.ds(my * CHUNK, CHUNK), :] = ystore[pl.ds(my * CHUNK, CHUNK), :]

        for p in range(N_DEV):
            @pl.when(p != my)
            def _():
                recv = pltpu.make_async_remote_copy(
                    src_ref=rs_buf.at[pl.ds(p * CHUNK, CHUNK)],
                    dst_ref=rs_buf.at[pl.ds(p * CHUNK, CHUNK)],
                    send_sem=rs_send_sems.at[0],
                    recv_sem=rs_recv_sems.at[p],
                    device_id=(p,),
                    device_id_type=pl.DeviceIdType.MESH,
                )
                recv.wait_recv()

        total = jnp.zeros((CHUNK, D), jnp.float32)
        for p in range(N_DEV):
            total = total + rs_buf[p * CHUNK:(p + 1) * CHUNK, :].astype(
                jnp.float32)
        out_ref[...] = total

        for rdma in ag_rdmas + rs_rdmas:
            rdma.wait_send()

    out = pl.pallas_call(
        body,
        out_shape=jax.ShapeDtypeStruct((CHUNK, D), jnp.float32),
        in_specs=[pl.BlockSpec(memory_space=pltpu.VMEM)] * 5,
        out_specs=pl.BlockSpec(memory_space=pltpu.VMEM),
        scratch_shapes=[
            pltpu.VMEM((SEQ, D), jnp.bfloat16),
            pltpu.VMEM((SEQ, D), jnp.bfloat16),
            pltpu.VMEM((SEQ, D), jnp.bfloat16),
            pltpu.VMEM((SEQ, D), jnp.bfloat16),
            pltpu.VMEM((SEQ, D), jnp.bfloat16),
            pltpu.VMEM((SEQ, D), jnp.bfloat16),
            pltpu.SemaphoreType.DMA((N_DEV - 1,)),
            pltpu.SemaphoreType.DMA((N_DEV - 1,)),
            pltpu.SemaphoreType.DMA((N_DEV - 1,)),
            pltpu.SemaphoreType.DMA((N_DEV,)),
        ],
        compiler_params=pltpu.CompilerParams(
            collective_id=0,
            vmem_limit_bytes=100 * 1024 * 1024,
        ),
    )(x2, Wq, Wo, Wk, Wv)
    return out.reshape(1, CHUNK, D)


# device time: 50812 ns/iter; 1.5200x vs baseline; 1.2555x over previous
import jax
import jax.numpy as jnp
from jax import lax
from jax.experimental import pallas as pl
from jax.experimental.pallas import tpu as pltpu

N_DEV = 4
SEQ = 1024
CHUNK = 256
HALF = 128
D = 1024
HEADS = 8
DH = 128
SCALE = 0.08838834764831843
LOG2E = 1.4426950408889634


def kernel(x, Wq, Wo, Wk, Wv):
    x2 = x.reshape(CHUNK, D)

    def body(x_ref, wq_ref, wo_ref, wk_ref, wv_ref, out_ref,
             x0, x1, x2, x3, qbuf, kbuf, vbuf, ystore, rs_buf,
             ag_send_sems, ag_recv_sems, rs_send_sems, rs_recv_sems):
        xslot = [x0, x1, x2, x3]
        my = lax.axis_index("i")
        left = (my - 1) % N_DEV
        right = (my + 1) % N_DEV

        barrier_sem = pltpu.get_barrier_semaphore()
        for nbr in (left, right):
            pl.semaphore_signal(barrier_sem, inc=1, device_id=(nbr,),
                                device_id_type=pl.DeviceIdType.MESH)
        pl.semaphore_wait(barrier_sem, 2)

        acc, lsum = {}, {}

        def q_proj(u):
            rows = pl.ds(u * CHUNK, CHUNK)
            xc = xslot[u][...]
            qbuf[rows, :] = (
                jnp.dot(xc, wq, preferred_element_type=jnp.float32)
                * (SCALE * LOG2E)
            ).astype(jnp.bfloat16)

        def kv_proj(u):
            rows = pl.ds(u * CHUNK, CHUNK)
            xc = xslot[u][...]
            kbuf[rows, :] = jnp.dot(
                xc, wk, preferred_element_type=jnp.float32
            ).astype(jnp.bfloat16)
            vbuf[rows, :] = jnp.dot(
                xc, wv, preferred_element_type=jnp.float32
            ).astype(jnp.bfloat16)

        def qkv(u):
            q_proj(u)
            kv_proj(u)

        def new_q_block(u, nkv):
            for hh in range(HEADS):
                sl = slice(hh * DH, (hh + 1) * DH)
                s = lax.dot_general(
                    qbuf[u * CHUNK:(u + 1) * CHUNK, sl],
                    kbuf[0:nkv * CHUNK, sl],
                    (((1,), (1,)), ((), ())),
                    preferred_element_type=jnp.float32)
                p = jnp.exp2(s)
                dl = jnp.sum(p, axis=-1, keepdims=True)
                do = jnp.dot(p.astype(jnp.bfloat16),
                             vbuf[0:nkv * CHUNK, sl],
                             preferred_element_type=jnp.float32)
                if (u, hh) in acc:
                    acc[(u, hh)] = acc[(u, hh)] + do
                    lsum[(u, hh)] = lsum[(u, hh)] + dl
                else:
                    acc[(u, hh)] = do
                    lsum[(u, hh)] = dl

        def old_q_block(nq, t):
            for hh in range(HEADS):
                sl = slice(hh * DH, (hh + 1) * DH)
                s = lax.dot_general(
                    qbuf[0:nq * CHUNK, sl],
                    kbuf[t * CHUNK:(t + 1) * CHUNK, sl],
                    (((1,), (1,)), ((), ())),
                    preferred_element_type=jnp.float32)
                p = jnp.exp2(s)
                dl = jnp.sum(p, axis=-1, keepdims=True)
                do = jnp.dot(p.astype(jnp.bfloat16),
                             vbuf[t * CHUNK:(t + 1) * CHUNK, sl],
                             preferred_element_type=jnp.float32)
                for u in range(nq):
                    rows = slice(u * CHUNK, (u + 1) * CHUNK)
                    acc[(u, hh)] = acc[(u, hh)] + do[rows]
                    lsum[(u, hh)] = lsum[(u, hh)] + dl[rows]

        def single_q_block(u, t):
            for hh in range(HEADS):
                sl = slice(hh * DH, (hh + 1) * DH)
                s = lax.dot_general(
                    qbuf[u * CHUNK:(u + 1) * CHUNK, sl],
                    kbuf[t * CHUNK:(t + 1) * CHUNK, sl],
                    (((1,), (1,)), ((), ())),
                    preferred_element_type=jnp.float32)
                p = jnp.exp2(s)
                acc[(u, hh)] = acc[(u, hh)] + jnp.dot(
                    p.astype(jnp.bfloat16),
                    vbuf[t * CHUNK:(t + 1) * CHUNK, sl],
                    preferred_element_type=jnp.float32)
                lsum[(u, hh)] = lsum[(u, hh)] + jnp.sum(
                    p, axis=-1, keepdims=True)

        def finalize(u):
            y = jnp.zeros((CHUNK, D), jnp.float32)
            for hh in range(HEADS):
                sl = slice(hh * DH, (hh + 1) * DH)
                o = (acc[(u, hh)] * (1.0 / lsum[(u, hh)])).astype(
                    jnp.bfloat16)
                y = y + jnp.dot(o, wo[sl, :],
                                preferred_element_type=jnp.float32)
            return y

        x0[...] = x_ref[...].astype(jnp.bfloat16)

        def ag_copy(src_slot, dst_slot, dev, sem_idx):
            return pltpu.make_async_remote_copy(
                src_ref=xslot[src_slot],
                dst_ref=xslot[dst_slot],
                send_sem=ag_send_sems.at[sem_idx],
                recv_sem=ag_recv_sems.at[sem_idx],
                device_id=(dev,),
                device_id_type=pl.DeviceIdType.MESH,
            )

        r_R = ag_copy(0, 1, right, 0)
        r_L = ag_copy(0, 2, left, 1)
        r_R.start()
        r_L.start()
        wq = wq_ref[...].astype(jnp.bfloat16)
        wk = wk_ref[...].astype(jnp.bfloat16)
        wv = wv_ref[...].astype(jnp.bfloat16)
        qkv(0)
        new_q_block(0, 1)
        r_R.wait_recv()
        qkv(1)
        old_q_block(1, 1)
        new_q_block(1, 2)
        r_L.wait_recv()
        r_D = ag_copy(2, 3, left, 2)
        r_D.start()
        wo = wo_ref[...].astype(jnp.bfloat16)
        qkv(2)
        old_q_block(2, 2)
        new_q_block(2, 3)
        r_D.wait_recv()
        rd = {0: r_R, 1: r_L, 2: r_D}

        rs_rdmas = []
        slot_dev = {1: left, 2: right, 3: (my + 2) % N_DEV}

        def send(u, d_idx):
            ystore[pl.ds(u * CHUNK, CHUNK), :] = finalize(u).astype(
                jnp.bfloat16)
            rdma = pltpu.make_async_remote_copy(
                src_ref=ystore.at[pl.ds(u * CHUNK, CHUNK)],
                dst_ref=rs_buf.at[pl.ds(my * CHUNK, CHUNK)],
                send_sem=rs_send_sems.at[d_idx],
                recv_sem=rs_recv_sems.at[my],
                device_id=(slot_dev[u],),
                device_id_type=pl.DeviceIdType.MESH,
            )
            rdma.start()
            rs_rdmas.append(rdma)

        kv_proj(3)
        single_q_block(1, 3)
        send(1, 0)
        single_q_block(2, 3)
        send(2, 1)
        q_proj(3)
        new_q_block(3, 4)
        send(3, 2)
        single_q_block(0, 3)
        rs_buf[pl.ds(my * CHUNK, CHUNK), :] = finalize(0).astype(jnp.bfloat16)

        total = rs_buf[pl.ds(my * CHUNK, CHUNK), :].astype(jnp.float32)
        for p in range(N_DEV):
            @pl.when(p != my)
            def _():
                recv = pltpu.make_async_remote_copy(
                    src_ref=rs_buf.at[pl.ds(p * CHUNK, CHUNK)],
                    dst_ref=rs_buf.at[pl.ds(p * CHUNK, CHUNK)],
                    send_sem=rs_send_sems.at[0],
                    recv_sem=rs_recv_sems.at[p],
                    device_id=(p,),
                    device_id_type=pl.DeviceIdType.MESH,
                )
                recv.wait_recv()
            cond = (p != my).astype(jnp.float32)
            total = total + cond * rs_buf[
                p * CHUNK:(p + 1) * CHUNK, :].astype(jnp.float32)
        out_ref[...] = total

        for r in rd.values():
            r.wait_send()
        for rdma in rs_rdmas:
            rdma.wait_send()

    out = pl.pallas_call(
        body,
        out_shape=jax.ShapeDtypeStruct((CHUNK, D), jnp.float32),
        in_specs=[pl.BlockSpec(memory_space=pltpu.VMEM)] * 5,
        out_specs=pl.BlockSpec(memory_space=pltpu.VMEM),
        scratch_shapes=[
            pltpu.VMEM((CHUNK, D), jnp.bfloat16),
            pltpu.VMEM((CHUNK, D), jnp.bfloat16),
            pltpu.VMEM((CHUNK, D), jnp.bfloat16),
            pltpu.VMEM((CHUNK, D), jnp.bfloat16),
            pltpu.VMEM((SEQ, D), jnp.bfloat16),
            pltpu.VMEM((SEQ, D), jnp.bfloat16),
            pltpu.VMEM((SEQ, D), jnp.bfloat16),
            pltpu.VMEM((SEQ, D), jnp.bfloat16),
            pltpu.VMEM((SEQ, D), jnp.bfloat16),
            pltpu.SemaphoreType.DMA((3,)),
            pltpu.SemaphoreType.DMA((3,)),
            pltpu.SemaphoreType.DMA((N_DEV - 1,)),
            pltpu.SemaphoreType.DMA((N_DEV,)),
        ],
        compiler_params=pltpu.CompilerParams(
            collective_id=0,
            vmem_limit_bytes=100 * 1024 * 1024,
        ),
    )(x2, Wq, Wo, Wk, Wv)
    return out.reshape(1, CHUNK, D)
